# Initial kernel scaffold; baseline (speedup 1.0000x reference)
import jax
import jax.numpy as jnp
from jax.experimental import pallas as pl


def kernel(node_features, mesh_edge_features, senders, receivers, params):
    raise NotImplementedError("write your pallas kernel here")


# speedup vs baseline: 3.0694x; 3.0694x over previous
import jax
import jax.numpy as jnp
from jax import lax
from jax.experimental import pallas as pl
from jax.experimental.pallas import tpu as pltpu
from jax.experimental.pallas import tpu_sc as plsc

_EPS = 1e-5
_CHUNK = 128
_NW = 32


def _relu(x):
    return jnp.maximum(x, 0.0)


def _dot(a, b):
    return jnp.dot(a, b, preferred_element_type=jnp.float32)


def _ln(h, g, b):
    mu = jnp.mean(h, axis=-1, keepdims=True)
    var = jnp.mean((h - mu) ** 2, axis=-1, keepdims=True)
    return (h - mu) * lax.rsqrt(var + _EPS) * g + b


def _enc_body(x, W1, b1, W2, b2, g, be, o):
    h = _relu(_dot(x[...], W1[...]) + b1[...])
    h = _relu(_dot(h, W2[...]) + b2[...])
    o[...] = _ln(h, g[...], be[...])


def _pre_body(x, Ws, Wr, b1, a, b):
    xv = x[...]
    a[...] = _dot(xv, Ws[...]) + b1[...]
    b[...] = _dot(xv, Wr[...])


def _edge_body(c, e, We, W2, b2, g, be, ne, eo):
    ev = e[...]
    h = _relu(c[...] + _dot(ev, We[...]))
    h = _relu(_dot(h, W2[...]) + b2[...])
    nev = _ln(h, g[...], be[...])
    ne[...] = nev
    eo[...] = ev + nev


def _node_body(x, p0, p1, Wn, Wa, b1, W2, b2, g, be, o):
    xv = x[...]
    aggr = p0[...] + p1[...]
    h = _relu(_dot(xv, Wn[...]) + _dot(aggr, Wa[...]) + b1[...])
    h = _relu(_dot(h, W2[...]) + b2[...])
    o[...] = xv + _ln(h, g[...], be[...])


def _dec_body(x, W1, b1, W2, b2, dt, o):
    h = _dot(x[...], W1[...]) + b1[...]
    h = h * jax.nn.sigmoid(h)
    o[...] = (_dot(h, W2[...]) + b2[...]) * dt[...]


def _pick_tr(rows):
    for t in (2000, 1000, 500, 200, 100, 8):
        if rows % t == 0:
            return t
    return rows


def _rowwise(body, tiled_in, const_in, out_widths, out_dtype=jnp.float32):
    rows = tiled_in[0].shape[0]
    tr = _pick_tr(rows)
    grid = rows // tr
    in_specs = (
        [pl.BlockSpec((tr, a.shape[1]), lambda i: (i, 0)) for a in tiled_in]
        + [pl.BlockSpec(a.shape, lambda i, _nd=a.ndim: (0,) * _nd) for a in const_in]
    )
    out_shape = [jax.ShapeDtypeStruct((rows, w), out_dtype) for w in out_widths]
    out_specs = [pl.BlockSpec((tr, w), lambda i: (i, 0)) for w in out_widths]
    outs = pl.pallas_call(
        body,
        grid=(grid,),
        in_specs=in_specs,
        out_specs=out_specs,
        out_shape=out_shape,
    )(*tiled_in, *const_in)
    return outs


def _make_gather(n, e, l):
    nchunk = e // _CHUNK
    iters = (nchunk + _NW - 1) // _NW
    mesh = plsc.VectorSubcoreMesh(core_axis_name="c", subcore_axis_name="s", num_cores=2, num_subcores=16)

    def body(a_hbm, b_hbm, snd_hbm, rcv_hbm, out_hbm, idx_s, idx_r, rows, sem):
        cid = lax.axis_index("c")
        sid = lax.axis_index("s")
        wid = sid * 2 + cid

        def step(k, carry):
            c = wid + _NW * k

            @pl.when(c < nchunk)
            def _():
                base = c * _CHUNK
                pltpu.sync_copy(snd_hbm.at[pl.ds(base, _CHUNK)], idx_s)
                pltpu.sync_copy(rcv_hbm.at[pl.ds(base, _CHUNK)], idx_r)
                pltpu.async_copy(a_hbm.at[idx_s], rows, sem).wait()
                pltpu.async_copy(b_hbm.at[idx_r], rows, sem, add=True).wait()
                pltpu.sync_copy(rows, out_hbm.at[pl.ds(base, _CHUNK), :])

            return carry

        lax.fori_loop(0, iters, step, 0)

    return pl.kernel(
        body,
        out_type=jax.ShapeDtypeStruct((e, l), jnp.float32),
        mesh=mesh,
        scratch_types=[
            pltpu.VMEM((_CHUNK,), jnp.int32),
            pltpu.VMEM((_CHUNK,), jnp.int32),
            pltpu.VMEM((_CHUNK, l), jnp.float32),
            pltpu.SemaphoreType.DMA,
        ],
    )


def _make_scatter(n, e, l):
    nchunk = e // _CHUNK
    iters = (nchunk + _NW - 1) // _NW
    rpt = ((n // 16) // 8) * 8
    last = n - 15 * rpt
    mesh = plsc.VectorSubcoreMesh(core_axis_name="c", subcore_axis_name="s", num_cores=2, num_subcores=16)

    def _striped_copy(sid, src, dst):
        @pl.when(sid < 15)
        def _():
            pltpu.sync_copy(src.at[pl.ds(sid * rpt, rpt), :],
                            dst.at[pl.ds(sid * rpt, rpt), :])

        @pl.when(sid == 15)
        def _():
            pltpu.sync_copy(src.at[pl.ds(15 * rpt, last), :],
                            dst.at[pl.ds(15 * rpt, last), :])

    def body(ne_hbm, rcv_hbm, zero_hbm, out_hbm, idx_r, rows, sem, acc):
        cid = lax.axis_index("c")
        sid = lax.axis_index("s")
        wid = sid * 2 + cid

        _striped_copy(sid, zero_hbm, acc)
        plsc.subcore_barrier()

        def step(k, carry):
            c = wid + _NW * k

            @pl.when(c < nchunk)
            def _():
                base = c * _CHUNK
                pltpu.sync_copy(rcv_hbm.at[pl.ds(base, _CHUNK)], idx_r)
                pltpu.sync_copy(ne_hbm.at[pl.ds(base, _CHUNK), :], rows)
                pltpu.sync_copy(rows, acc.at[idx_r], add=True)

            return carry

        lax.fori_loop(0, iters, step, 0)
        plsc.subcore_barrier()
        _striped_copy(sid, acc, out_hbm.at[cid])

    return pl.kernel(
        body,
        out_type=jax.ShapeDtypeStruct((2, n, l), jnp.float32),
        mesh=mesh,
        scratch_types=[
            pltpu.VMEM((_CHUNK,), jnp.int32),
            pltpu.VMEM((_CHUNK, l), jnp.float32),
            pltpu.SemaphoreType.DMA,
            pltpu.VMEM_SHARED((n, l), jnp.float32),
        ],
    )


def kernel(node_features, mesh_edge_features, senders, receivers, params):
    p = params
    n, l = node_features.shape
    e = senders.shape[0]
    s_blocks = p["blk_e_W1"].shape[0]

    def r2(v):
        return v.reshape(1, -1)

    node_lat, = _rowwise(
        _enc_body, [node_features],
        [p["enc_n_W1"], r2(p["enc_n_b1"]), p["enc_n_W2"], r2(p["enc_n_b2"]),
         r2(p["enc_n_g"]), r2(p["enc_n_be"])], [l])
    edge_lat, = _rowwise(
        _enc_body, [mesh_edge_features],
        [p["enc_e_W1"], r2(p["enc_e_b1"]), p["enc_e_W2"], r2(p["enc_e_b2"]),
         r2(p["enc_e_g"]), r2(p["enc_e_be"])], [l])

    gather = _make_gather(n, e, l)
    scatter = _make_scatter(n, e, l)
    zeros_nl = jnp.zeros((n, l), jnp.float32)

    for s in range(s_blocks):
        w1 = p["blk_e_W1"][s]
        a_tab, b_tab = _rowwise(
            _pre_body, [node_lat],
            [w1[:l], w1[l:2 * l], r2(p["blk_e_b1"][s])], [l, l])
        c = gather(a_tab, b_tab, senders, receivers)
        new_e, edge_lat = _rowwise(
            _edge_body, [c, edge_lat],
            [w1[2 * l:], p["blk_e_W2"][s], r2(p["blk_e_b2"][s]),
             r2(p["blk_e_g"][s]), r2(p["blk_e_be"][s])], [l, l])
        partials = scatter(new_e, receivers, zeros_nl)
        wn1 = p["blk_n_W1"][s]
        node_lat, = _rowwise(
            _node_body, [node_lat, partials[0], partials[1]],
            [wn1[:l], wn1[l:], r2(p["blk_n_b1"][s]), p["blk_n_W2"][s],
             r2(p["blk_n_b2"][s]), r2(p["blk_n_g"][s]), r2(p["blk_n_be"][s])],
            [l])

    tw_out = p["dec_b2"].shape[0]
    tw = 5
    out_c = tw_out // tw
    dt = jnp.repeat(jnp.arange(1, tw + 1), out_c).astype(jnp.float32)
    dec, = _rowwise(
        _dec_body, [node_lat],
        [p["dec_W1"], r2(p["dec_b1"]), p["dec_W2"], r2(p["dec_b2"]), r2(dt)],
        [tw_out])
    return dec.reshape(n, tw, out_c).transpose(1, 0, 2)


# speedup vs baseline: 4.3715x; 1.4242x over previous
import jax
import jax.numpy as jnp
from jax import lax
from jax.experimental import pallas as pl
from jax.experimental.pallas import tpu as pltpu
from jax.experimental.pallas import tpu_sc as plsc

_EPS = 1e-5
_CHUNK = 128
_NW = 32


def _relu(x):
    return jnp.maximum(x, 0.0)


def _dot(a, b):
    return jnp.dot(a, b, preferred_element_type=jnp.float32)


def _ln(h, g, b):
    mu = jnp.mean(h, axis=-1, keepdims=True)
    var = jnp.mean((h - mu) ** 2, axis=-1, keepdims=True)
    return (h - mu) * lax.rsqrt(var + _EPS) * g + b


def _enc_body(x, W1, b1, W2, b2, g, be, o):
    h = _relu(_dot(x[...], W1[...]) + b1[...])
    h = _relu(_dot(h, W2[...]) + b2[...])
    o[...] = _ln(h, g[...], be[...])


def _pre_body(x, Ws, Wr, b1, a, b):
    xv = x[...]
    a[...] = _dot(xv, Ws[...]) + b1[...]
    b[...] = _dot(xv, Wr[...])


def _edge_body(c, e, We, W2, b2, g, be, ne, eo):
    ev = e[...]
    h = _relu(c[...] + _dot(ev, We[...]))
    h = _relu(_dot(h, W2[...]) + b2[...])
    nev = _ln(h, g[...], be[...])
    ne[...] = nev
    eo[...] = ev + nev


def _node_body(x, p0, p1, Wn, Wa, b1, W2, b2, g, be, o):
    xv = x[...]
    aggr = p0[...] + p1[...]
    h = _relu(_dot(xv, Wn[...]) + _dot(aggr, Wa[...]) + b1[...])
    h = _relu(_dot(h, W2[...]) + b2[...])
    o[...] = xv + _ln(h, g[...], be[...])


def _dec_body(x, W1, b1, W2, b2, dt, o):
    h = _dot(x[...], W1[...]) + b1[...]
    h = h * jax.nn.sigmoid(h)
    o[...] = (_dot(h, W2[...]) + b2[...]) * dt[...]


def _pick_tr(rows):
    for t in (2000, 1000, 500, 200, 100, 8):
        if rows % t == 0:
            return t
    return rows


def _rowwise(body, tiled_in, const_in, out_widths, out_dtype=jnp.float32):
    rows = tiled_in[0].shape[0]
    tr = _pick_tr(rows)
    grid = rows // tr
    in_specs = (
        [pl.BlockSpec((tr, a.shape[1]), lambda i: (i, 0)) for a in tiled_in]
        + [pl.BlockSpec(a.shape, lambda i, _nd=a.ndim: (0,) * _nd) for a in const_in]
    )
    out_shape = [jax.ShapeDtypeStruct((rows, w), out_dtype) for w in out_widths]
    out_specs = [pl.BlockSpec((tr, w), lambda i: (i, 0)) for w in out_widths]
    outs = pl.pallas_call(
        body,
        grid=(grid,),
        in_specs=in_specs,
        out_specs=out_specs,
        out_shape=out_shape,
    )(*tiled_in, *const_in)
    return outs


_SS = 256
_NBUF = 3


def _make_gather(n, e, l):
    nch = e // _SS
    groups = (((nch + _NW - 1) // _NW) + _NBUF - 1) // _NBUF
    mesh = plsc.VectorSubcoreMesh(core_axis_name="c", subcore_axis_name="s", num_cores=2, num_subcores=16)

    def body(a_hbm, b_hbm, snd_hbm, rcv_hbm, out_hbm,
             idx_s, idx_r, rows, sem_idx, sem_g, sem_out):
        cid = lax.axis_index("c")
        sid = lax.axis_index("s")
        wid = sid * 2 + cid

        def issue_idx(k, b):
            c = wid + _NW * k

            @pl.when(c < nch)
            def _():
                base = c * _SS
                pltpu.async_copy(snd_hbm.at[pl.ds(base, _SS)], idx_s[b], sem_idx[b])
                pltpu.async_copy(rcv_hbm.at[pl.ds(base, _SS)], idx_r[b], sem_idx[b])

        for b in range(_NBUF):
            issue_idx(b, b)

        def step(k, b):
            c = wid + _NW * k

            @pl.when(c < nch)
            def _():
                base = c * _SS
                pltpu.make_async_copy(snd_hbm.at[pl.ds(base, _SS)], idx_s[b], sem_idx[b]).wait()
                pltpu.make_async_copy(rcv_hbm.at[pl.ds(base, _SS)], idx_r[b], sem_idx[b]).wait()

                @pl.when(k >= _NBUF)
                def _():
                    pltpu.make_async_copy(rows[b], out_hbm.at[pl.ds(base, _SS), :], sem_out[b]).wait()

                d0 = pltpu.async_copy(a_hbm.at[idx_s[b].at[pl.ds(0, _CHUNK)]],
                                      rows[b].at[pl.ds(0, _CHUNK), :], sem_g[b])
                d1 = pltpu.async_copy(a_hbm.at[idx_s[b].at[pl.ds(_CHUNK, _CHUNK)]],
                                      rows[b].at[pl.ds(_CHUNK, _CHUNK), :], sem_g[b])
                d0.wait()
                d1.wait()
                d2 = pltpu.async_copy(b_hbm.at[idx_r[b].at[pl.ds(0, _CHUNK)]],
                                      rows[b].at[pl.ds(0, _CHUNK), :], sem_g[b], add=True)
                d3 = pltpu.async_copy(b_hbm.at[idx_r[b].at[pl.ds(_CHUNK, _CHUNK)]],
                                      rows[b].at[pl.ds(_CHUNK, _CHUNK), :], sem_g[b], add=True)
                d2.wait()
                d3.wait()

                issue_idx(k + _NBUF, b)
                pltpu.async_copy(rows[b], out_hbm.at[pl.ds(base, _SS), :], sem_out[b])

        def group(g, carry):
            for b in range(_NBUF):
                step(g * _NBUF + b, b)
            return carry

        lax.fori_loop(0, groups, group, 0)
        for b in range(_NBUF):
            pltpu.make_async_copy(rows[b], out_hbm.at[pl.ds(0, _SS), :], sem_out[b]).wait()

    return pl.kernel(
        body,
        out_type=jax.ShapeDtypeStruct((e, l), jnp.float32),
        mesh=mesh,
        scratch_types=[
            [pltpu.VMEM((_SS,), jnp.int32) for _ in range(_NBUF)],
            [pltpu.VMEM((_SS,), jnp.int32) for _ in range(_NBUF)],
            [pltpu.VMEM((_SS, l), jnp.float32) for _ in range(_NBUF)],
            [pltpu.SemaphoreType.DMA for _ in range(_NBUF)],
            [pltpu.SemaphoreType.DMA for _ in range(_NBUF)],
            [pltpu.SemaphoreType.DMA for _ in range(_NBUF)],
        ],
    )


def _make_scatter(n, e, l):
    rpt = ((n // 16) // 8) * 8
    last = n - 15 * rpt
    mesh = plsc.VectorSubcoreMesh(core_axis_name="c", subcore_axis_name="s", num_cores=2, num_subcores=16)

    def _striped_copy(sid, src, dst):
        @pl.when(sid < 15)
        def _():
            pltpu.sync_copy(src.at[pl.ds(sid * rpt, rpt), :],
                            dst.at[pl.ds(sid * rpt, rpt), :])

        @pl.when(sid == 15)
        def _():
            pltpu.sync_copy(src.at[pl.ds(15 * rpt, last), :],
                            dst.at[pl.ds(15 * rpt, last), :])

    nch = e // _CHUNK
    nbuf = 2
    groups = (((nch + _NW - 1) // _NW) + nbuf - 1) // nbuf

    def body(ne_hbm, rcv_hbm, zero_hbm, out_hbm, idx_r, rows, sem_in, sem_sc, acc):
        cid = lax.axis_index("c")
        sid = lax.axis_index("s")
        wid = sid * 2 + cid

        def issue_in(k, b):
            c = wid + _NW * k

            @pl.when(c < nch)
            def _():
                base = c * _CHUNK
                pltpu.async_copy(rcv_hbm.at[pl.ds(base, _CHUNK)], idx_r[b], sem_in[b])
                pltpu.async_copy(ne_hbm.at[pl.ds(base, _CHUNK), :], rows[b], sem_in[b])

        for b in range(nbuf):
            issue_in(b, b)

        _striped_copy(sid, zero_hbm, acc)
        plsc.subcore_barrier()

        def step(k, b):
            c = wid + _NW * k

            @pl.when(c < nch)
            def _():
                base = c * _CHUNK
                pltpu.make_async_copy(rcv_hbm.at[pl.ds(base, _CHUNK)], idx_r[b], sem_in[b]).wait()
                pltpu.make_async_copy(ne_hbm.at[pl.ds(base, _CHUNK), :], rows[b], sem_in[b]).wait()
                pltpu.async_copy(rows[b], acc.at[idx_r[b]], sem_sc[b], add=True).wait()
                issue_in(k + nbuf, b)

        def group(g, carry):
            for b in range(nbuf):
                step(g * nbuf + b, b)
            return carry

        lax.fori_loop(0, groups, group, 0)
        plsc.subcore_barrier()
        _striped_copy(sid, acc, out_hbm.at[cid])

    return pl.kernel(
        body,
        out_type=jax.ShapeDtypeStruct((2, n, l), jnp.float32),
        mesh=mesh,
        scratch_types=[
            [pltpu.VMEM((_CHUNK,), jnp.int32) for _ in range(nbuf)],
            [pltpu.VMEM((_CHUNK, l), jnp.float32) for _ in range(nbuf)],
            [pltpu.SemaphoreType.DMA for _ in range(nbuf)],
            [pltpu.SemaphoreType.DMA for _ in range(nbuf)],
            pltpu.VMEM_SHARED((n, l), jnp.float32),
        ],
    )


def kernel(node_features, mesh_edge_features, senders, receivers, params):
    p = params
    n, l = node_features.shape
    e = senders.shape[0]
    s_blocks = p["blk_e_W1"].shape[0]

    def r2(v):
        return v.reshape(1, -1)

    node_lat, = _rowwise(
        _enc_body, [node_features],
        [p["enc_n_W1"], r2(p["enc_n_b1"]), p["enc_n_W2"], r2(p["enc_n_b2"]),
         r2(p["enc_n_g"]), r2(p["enc_n_be"])], [l])
    edge_lat, = _rowwise(
        _enc_body, [mesh_edge_features],
        [p["enc_e_W1"], r2(p["enc_e_b1"]), p["enc_e_W2"], r2(p["enc_e_b2"]),
         r2(p["enc_e_g"]), r2(p["enc_e_be"])], [l])

    gather = _make_gather(n, e, l)
    scatter = _make_scatter(n, e, l)
    zeros_nl = jnp.zeros((n, l), jnp.float32)

    for s in range(s_blocks):
        w1 = p["blk_e_W1"][s]
        a_tab, b_tab = _rowwise(
            _pre_body, [node_lat],
            [w1[:l], w1[l:2 * l], r2(p["blk_e_b1"][s])], [l, l])
        c = gather(a_tab, b_tab, senders, receivers)
        new_e, edge_lat = _rowwise(
            _edge_body, [c, edge_lat],
            [w1[2 * l:], p["blk_e_W2"][s], r2(p["blk_e_b2"][s]),
             r2(p["blk_e_g"][s]), r2(p["blk_e_be"][s])], [l, l])
        partials = scatter(new_e, receivers, zeros_nl)
        wn1 = p["blk_n_W1"][s]
        node_lat, = _rowwise(
            _node_body, [node_lat, partials[0], partials[1]],
            [wn1[:l], wn1[l:], r2(p["blk_n_b1"][s]), p["blk_n_W2"][s],
             r2(p["blk_n_b2"][s]), r2(p["blk_n_g"][s]), r2(p["blk_n_be"][s])],
            [l])

    tw_out = p["dec_b2"].shape[0]
    tw = 5
    out_c = tw_out // tw
    dt = jnp.repeat(jnp.arange(1, tw + 1), out_c).astype(jnp.float32)
    dec, = _rowwise(
        _dec_body, [node_lat],
        [p["dec_W1"], r2(p["dec_b1"]), p["dec_W2"], r2(p["dec_b2"]), r2(dt)],
        [tw_out])
    return dec.reshape(n, tw, out_c).transpose(1, 0, 2)
